# baseline (speedup 1.0000x reference)
import functools

import jax
import jax.numpy as jnp
from jax import lax
from jax.experimental import pallas as pl
from jax.experimental.pallas import tpu as pltpu
from jax.experimental.pallas import tpu_sc as plsc

_MAXLEN = 8192
_D = 128
_SC_ROWS = _MAXLEN // 2
_TC_ROWS = _MAXLEN - _SC_ROWS

_info = plsc.get_sparse_core_info()
_NC = _info.num_cores
_NS = _info.num_subcores
_L = _info.num_lanes
_NW = _NC * _NS
_B_PER_W = _SC_ROWS // _NW

_mesh = plsc.VectorSubcoreMesh(core_axis_name="c", subcore_axis_name="s")


@functools.partial(
    pl.kernel,
    mesh=_mesh,
    out_type=jax.ShapeDtypeStruct((_SC_ROWS, _D), jnp.float32),
    scratch_types=[
        pltpu.VMEM((_B_PER_W,), jnp.int32),
        pltpu.VMEM((_B_PER_W, _D), jnp.float32),
        pltpu.SemaphoreType.DMA,
        pltpu.SemaphoreType.DMA,
    ],
)
def _pos_embed_gather_sc(table_hbm, out_hbm, idx_v, rows_v, gsem, ssem):
    wid = lax.axis_index("s") * _NC + lax.axis_index("c")
    base = wid * _B_PER_W

    def _fill(i, carry):
        idx_v[pl.ds(i * _L, _L)] = lax.iota(jnp.int32, _L) + (base + i * _L)
        return carry

    lax.fori_loop(0, _B_PER_W // _L, _fill, 0)

    pltpu.async_copy(table_hbm.at[idx_v], rows_v, gsem).wait()
    pltpu.async_copy(rows_v, out_hbm.at[pl.ds(base, _B_PER_W)], ssem).wait()


def _tc_copy_body(src_ref, out_ref):
    out_ref[...] = src_ref[...]


_TC_BLOCK = 512


def _pos_embed_tc_half(table):
    return pl.pallas_call(
        _tc_copy_body,
        grid=(_TC_ROWS // _TC_BLOCK,),
        in_specs=[
            pl.BlockSpec((_TC_BLOCK, _D), lambda i: (i + _SC_ROWS // _TC_BLOCK, 0))
        ],
        out_specs=pl.BlockSpec((_TC_BLOCK, _D), lambda i: (i, 0)),
        out_shape=jax.ShapeDtypeStruct((_TC_ROWS, _D), jnp.float32),
    )(table)


def kernel(x, pos_table):
    del x
    lo = _pos_embed_gather_sc(pos_table)
    hi = _pos_embed_tc_half(pos_table)
    return jnp.concatenate([lo, hi], axis=0)


# speedup vs baseline: 1.1677x; 1.1677x over previous
import functools

import jax
import jax.numpy as jnp
from jax import lax
from jax.experimental import pallas as pl
from jax.experimental.pallas import tpu as pltpu
from jax.experimental.pallas import tpu_sc as plsc

_MAXLEN = 8192
_D = 128

_info = plsc.get_sparse_core_info()
_NC = _info.num_cores
_NS = _info.num_subcores
_L = _info.num_lanes
_NW = _NC * _NS
_B_PER_W = _MAXLEN // _NW
_CHUNK = 128
_NCHUNK = _B_PER_W // _CHUNK

_mesh = plsc.VectorSubcoreMesh(core_axis_name="c", subcore_axis_name="s")


@functools.partial(
    pl.kernel,
    mesh=_mesh,
    out_type=jax.ShapeDtypeStruct((_MAXLEN, _D), jnp.float32),
    scratch_types=[
        pltpu.VMEM((_NCHUNK, _CHUNK), jnp.int32),
        pltpu.VMEM((_B_PER_W, _D), jnp.float32),
        pltpu.SemaphoreType.DMA,
        pltpu.SemaphoreType.DMA,
    ],
)
def _pos_embed_gather(table_hbm, out_hbm, idx_v, rows_v, gsem, ssem):
    wid = lax.axis_index("s") * _NC + lax.axis_index("c")
    base = wid * _B_PER_W

    def _fill(i, carry):
        idx_v[i // (_CHUNK // _L), pl.ds((i % (_CHUNK // _L)) * _L, _L)] = (
            lax.iota(jnp.int32, _L) + (base + i * _L)
        )
        return carry

    lax.fori_loop(0, _B_PER_W // _L, _fill, 0)

    gathers = [
        pltpu.async_copy(
            table_hbm.at[idx_v.at[j]], rows_v.at[pl.ds(j * _CHUNK, _CHUNK)], gsem
        )
        for j in range(_NCHUNK)
    ]
    for g in gathers:
        g.wait()
    pltpu.async_copy(rows_v, out_hbm.at[pl.ds(base, _B_PER_W)], ssem).wait()


def kernel(x, pos_table):
    del x
    return _pos_embed_gather(pos_table)
